# initial kernel scaffold (unmeasured)
import jax
import jax.numpy as jnp
from jax import lax
from jax.experimental import pallas as pl
from jax.experimental.pallas import tpu as pltpu

N_DEV = 32


def kernel(x, w_mat):
    m_per, k = x.shape
    _, n_per = w_mat.shape
    half = m_per // 2

    def body(x_ref, w_ref, out_ref, cw_ref, ccw_ref,
             cw_send, cw_recv, ccw_send, ccw_recv):
        my = lax.axis_index("i")
        left = lax.rem(my - 1 + N_DEV, N_DEV)
        right = lax.rem(my + 1, N_DEV)

        barrier = pltpu.get_barrier_semaphore()
        for nbr in (left, right):
            pl.semaphore_signal(
                barrier, inc=1,
                device_id=(nbr,), device_id_type=pl.DeviceIdType.MESH,
            )
        pl.semaphore_wait(barrier, 2)

        w = w_ref[:, :]

        def silu_store(rows, row_start):
            y = jnp.dot(rows, w, preferred_element_type=jnp.float32)
            out_ref[pl.ds(row_start, y.shape[0]), :] = y * jax.nn.sigmoid(y)

        silu_store(x_ref[:, :], my * m_per)

        cw_ref[0, :, :] = x_ref[:half, :]
        ccw_ref[0, :, :] = x_ref[half:, :]

        for h in range(N_DEV - 1):
            s, r = h % 2, (h + 1) % 2
            cw = pltpu.make_async_remote_copy(
                src_ref=cw_ref.at[s], dst_ref=cw_ref.at[r],
                send_sem=cw_send.at[s], recv_sem=cw_recv.at[r],
                device_id=(right,), device_id_type=pl.DeviceIdType.MESH,
            )
            ccw = pltpu.make_async_remote_copy(
                src_ref=ccw_ref.at[s], dst_ref=ccw_ref.at[r],
                send_sem=ccw_send.at[s], recv_sem=ccw_recv.at[r],
                device_id=(left,), device_id_type=pl.DeviceIdType.MESH,
            )
            cw.start()
            ccw.start()
            cw.wait()
            ccw.wait()

            o_cw = lax.rem(my - (h + 1) + N_DEV, N_DEV)
            o_ccw = lax.rem(my + (h + 1), N_DEV)
            silu_store(cw_ref[r], o_cw * m_per)
            silu_store(ccw_ref[r], o_ccw * m_per + half)

    out_shape = jax.ShapeDtypeStruct((N_DEV * m_per, n_per), jnp.float32)
    return pl.pallas_call(
        body,
        out_shape=out_shape,
        in_specs=[
            pl.BlockSpec(memory_space=pltpu.VMEM),
            pl.BlockSpec(memory_space=pltpu.VMEM),
        ],
        out_specs=pl.BlockSpec(memory_space=pltpu.VMEM),
        scratch_shapes=[
            pltpu.VMEM((2, half, k), jnp.float32),
            pltpu.VMEM((2, half, k), jnp.float32),
            pltpu.SemaphoreType.DMA((2,)),
            pltpu.SemaphoreType.DMA((2,)),
            pltpu.SemaphoreType.DMA((2,)),
            pltpu.SemaphoreType.DMA((2,)),
        ],
        compiler_params=pltpu.CompilerParams(collective_id=0),
    )(x, w_mat)


# baseline (device time: 1509768 ns/iter reference)
import jax
import jax.numpy as jnp
from jax import lax
from jax.experimental import pallas as pl
from jax.experimental.pallas import tpu as pltpu

N_DEV = 32

PERM = [1, 2, 5, 6, 14, 13, 10, 9, 17, 18, 21, 22, 30, 29, 26, 25,
        24, 27, 28, 31, 23, 20, 19, 16, 8, 11, 12, 15, 7, 4, 3, 0]
CPOS = [0] * N_DEV
for _j, _l in enumerate(PERM):
    CPOS[_l] = _j


def kernel(x, w_mat):
    m_per, k = x.shape
    _, n_per = w_mat.shape
    half = m_per // 2

    perm_arr = jnp.asarray(PERM, dtype=jnp.int32)
    cpos_arr = jnp.asarray(CPOS, dtype=jnp.int32)

    def body(perm_ref, cpos_ref, x_ref, w_ref, out_ref, cw_ref, ccw_ref,
             cw_send, cw_recv, ccw_send, ccw_recv, cw_credit, ccw_credit):
        my = lax.axis_index("i")
        cp = cpos_ref[my]
        right = perm_ref[lax.rem(cp + 1, N_DEV)]
        left = perm_ref[lax.rem(cp + N_DEV - 1, N_DEV)]

        barrier = pltpu.get_barrier_semaphore()
        for nbr in (left, right):
            pl.semaphore_signal(
                barrier, inc=1,
                device_id=(nbr,), device_id_type=pl.DeviceIdType.MESH,
            )
        pl.semaphore_wait(barrier, 2)

        w = w_ref[:, :]

        def silu_store(rows, row_start):
            y = jnp.dot(rows, w, preferred_element_type=jnp.float32)
            out_ref[pl.ds(row_start, y.shape[0]), :] = y * jax.nn.sigmoid(y)

        silu_store(x_ref[:, :], my * m_per)

        cw_ref[0, :, :] = x_ref[:half, :]
        ccw_ref[0, :, :] = x_ref[half:, :]

        for h in range(N_DEV - 1):
            s, r = h % 2, (h + 1) % 2
            if h >= 2:
                pl.semaphore_wait(cw_credit, 1)
                pl.semaphore_wait(ccw_credit, 1)
            cw = pltpu.make_async_remote_copy(
                src_ref=cw_ref.at[s], dst_ref=cw_ref.at[r],
                send_sem=cw_send.at[s], recv_sem=cw_recv.at[r],
                device_id=(right,), device_id_type=pl.DeviceIdType.MESH,
            )
            ccw = pltpu.make_async_remote_copy(
                src_ref=ccw_ref.at[s], dst_ref=ccw_ref.at[r],
                send_sem=ccw_send.at[s], recv_sem=ccw_recv.at[r],
                device_id=(left,), device_id_type=pl.DeviceIdType.MESH,
            )
            cw.start()
            ccw.start()
            cw.wait()
            ccw.wait()
            if h <= N_DEV - 4:
                pl.semaphore_signal(
                    cw_credit, inc=1,
                    device_id=(left,), device_id_type=pl.DeviceIdType.MESH,
                )
                pl.semaphore_signal(
                    ccw_credit, inc=1,
                    device_id=(right,), device_id_type=pl.DeviceIdType.MESH,
                )

            o_cw = perm_ref[lax.rem(cp + N_DEV - 1 - h, N_DEV)]
            o_ccw = perm_ref[lax.rem(cp + 1 + h, N_DEV)]
            silu_store(cw_ref[r], o_cw * m_per)
            silu_store(ccw_ref[r], o_ccw * m_per + half)

    out_shape = jax.ShapeDtypeStruct((N_DEV * m_per, n_per), jnp.float32)
    return pl.pallas_call(
        body,
        out_shape=out_shape,
        in_specs=[
            pl.BlockSpec(memory_space=pltpu.SMEM),
            pl.BlockSpec(memory_space=pltpu.SMEM),
            pl.BlockSpec(memory_space=pltpu.VMEM),
            pl.BlockSpec(memory_space=pltpu.VMEM),
        ],
        out_specs=pl.BlockSpec(memory_space=pltpu.VMEM),
        scratch_shapes=[
            pltpu.VMEM((2, half, k), jnp.float32),
            pltpu.VMEM((2, half, k), jnp.float32),
            pltpu.SemaphoreType.DMA((2,)),
            pltpu.SemaphoreType.DMA((2,)),
            pltpu.SemaphoreType.DMA((2,)),
            pltpu.SemaphoreType.DMA((2,)),
            pltpu.SemaphoreType.REGULAR,
            pltpu.SemaphoreType.REGULAR,
        ],
        compiler_params=pltpu.CompilerParams(collective_id=0),
    )(perm_arr, cpos_arr, x, w_mat)


# device time: 1485232 ns/iter; 1.0165x vs baseline; 1.0165x over previous
import jax
import jax.numpy as jnp
from jax import lax
from jax.experimental import pallas as pl
from jax.experimental.pallas import tpu as pltpu

N_DEV = 32
SLOTS = 3

PERM = [1, 2, 5, 6, 14, 13, 10, 9, 17, 18, 21, 22, 30, 29, 26, 25,
        24, 27, 28, 31, 23, 20, 19, 16, 8, 11, 12, 15, 7, 4, 3, 0]
CPOS = [0] * N_DEV
for _j, _l in enumerate(PERM):
    CPOS[_l] = _j


def kernel(x, w_mat):
    m_per, k = x.shape
    _, n_per = w_mat.shape
    half = m_per // 2

    perm_arr = jnp.asarray(PERM, dtype=jnp.int32)
    cpos_arr = jnp.asarray(CPOS, dtype=jnp.int32)

    def body(perm_ref, cpos_ref, x_ref, w_ref, out_ref, cw_ref, ccw_ref,
             cw_send, cw_recv, ccw_send, ccw_recv, cw_credit, ccw_credit):
        my = lax.axis_index("i")
        cp = cpos_ref[my]
        right = perm_ref[lax.rem(cp + 1, N_DEV)]
        left = perm_ref[lax.rem(cp + N_DEV - 1, N_DEV)]

        barrier = pltpu.get_barrier_semaphore()
        for nbr in (left, right):
            pl.semaphore_signal(
                barrier, inc=1,
                device_id=(nbr,), device_id_type=pl.DeviceIdType.MESH,
            )
        pl.semaphore_wait(barrier, 2)

        w = w_ref[:, :]

        def silu_store(rows, row_start):
            y = jnp.dot(rows, w, preferred_element_type=jnp.float32)
            out_ref[pl.ds(row_start, y.shape[0]), :] = y * jax.nn.sigmoid(y)

        def make_cw(h):
            return pltpu.make_async_remote_copy(
                src_ref=cw_ref.at[h % SLOTS],
                dst_ref=cw_ref.at[(h + 1) % SLOTS],
                send_sem=cw_send.at[h % SLOTS],
                recv_sem=cw_recv.at[(h + 1) % SLOTS],
                device_id=(right,), device_id_type=pl.DeviceIdType.MESH,
            )

        def make_ccw(h):
            return pltpu.make_async_remote_copy(
                src_ref=ccw_ref.at[h % SLOTS],
                dst_ref=ccw_ref.at[(h + 1) % SLOTS],
                send_sem=ccw_send.at[h % SLOTS],
                recv_sem=ccw_recv.at[(h + 1) % SLOTS],
                device_id=(left,), device_id_type=pl.DeviceIdType.MESH,
            )

        cw_ref[0, :, :] = x_ref[:half, :]
        ccw_ref[0, :, :] = x_ref[half:, :]
        cw_desc = [None] * (N_DEV - 1)
        ccw_desc = [None] * (N_DEV - 1)
        cw_desc[0] = make_cw(0)
        ccw_desc[0] = make_ccw(0)
        cw_desc[0].start()
        ccw_desc[0].start()
        silu_store(x_ref[:, :], my * m_per)

        for h in range(N_DEV - 1):
            slot = (h + 1) % SLOTS
            cw_desc[h].wait_recv()
            ccw_desc[h].wait_recv()
            if h >= 1:
                cw_desc[h - 1].wait_send()
                ccw_desc[h - 1].wait_send()
                if h <= N_DEV - 3:
                    pl.semaphore_signal(
                        cw_credit, inc=1,
                        device_id=(left,), device_id_type=pl.DeviceIdType.MESH,
                    )
                    pl.semaphore_signal(
                        ccw_credit, inc=1,
                        device_id=(right,), device_id_type=pl.DeviceIdType.MESH,
                    )
            if h < N_DEV - 2:
                if h + 1 >= 2:
                    pl.semaphore_wait(cw_credit, 1)
                    pl.semaphore_wait(ccw_credit, 1)
                cw_desc[h + 1] = make_cw(h + 1)
                ccw_desc[h + 1] = make_ccw(h + 1)
                cw_desc[h + 1].start()
                ccw_desc[h + 1].start()

            o_cw = perm_ref[lax.rem(cp + N_DEV - 1 - h, N_DEV)]
            o_ccw = perm_ref[lax.rem(cp + 1 + h, N_DEV)]
            silu_store(cw_ref[slot], o_cw * m_per)
            silu_store(ccw_ref[slot], o_ccw * m_per + half)

        cw_desc[N_DEV - 2].wait_send()
        ccw_desc[N_DEV - 2].wait_send()

    out_shape = jax.ShapeDtypeStruct((N_DEV * m_per, n_per), jnp.float32)
    return pl.pallas_call(
        body,
        out_shape=out_shape,
        in_specs=[
            pl.BlockSpec(memory_space=pltpu.SMEM),
            pl.BlockSpec(memory_space=pltpu.SMEM),
            pl.BlockSpec(memory_space=pltpu.VMEM),
            pl.BlockSpec(memory_space=pltpu.VMEM),
        ],
        out_specs=pl.BlockSpec(memory_space=pltpu.VMEM),
        scratch_shapes=[
            pltpu.VMEM((SLOTS, half, k), jnp.float32),
            pltpu.VMEM((SLOTS, half, k), jnp.float32),
            pltpu.SemaphoreType.DMA((SLOTS,)),
            pltpu.SemaphoreType.DMA((SLOTS,)),
            pltpu.SemaphoreType.DMA((SLOTS,)),
            pltpu.SemaphoreType.DMA((SLOTS,)),
            pltpu.SemaphoreType.REGULAR,
            pltpu.SemaphoreType.REGULAR,
        ],
        compiler_params=pltpu.CompilerParams(collective_id=0),
    )(perm_arr, cpos_arr, x, w_mat)


# device time: 1463578 ns/iter; 1.0316x vs baseline; 1.0148x over previous
import jax
import jax.numpy as jnp
from jax import lax
from jax.experimental import pallas as pl
from jax.experimental.pallas import tpu as pltpu

N_DEV = 32
SLOTS = 3

PERM = [1, 2, 5, 6, 14, 13, 10, 9, 17, 18, 21, 22, 30, 29, 26, 25,
        24, 27, 28, 31, 23, 20, 19, 16, 8, 11, 12, 15, 7, 4, 3, 0]
CPOS = [0] * N_DEV
for _j, _l in enumerate(PERM):
    CPOS[_l] = _j


def kernel(x, w_mat):
    m_per, k = x.shape
    _, n_per = w_mat.shape
    half = m_per // 2

    perm_arr = jnp.asarray(PERM, dtype=jnp.int32)
    cpos_arr = jnp.asarray(CPOS, dtype=jnp.int32)

    def body(perm_ref, cpos_ref, x_ref, w_ref, out_ref, cw_ref, ccw_ref,
             cw_send, cw_recv, ccw_send, ccw_recv, cw_credit, ccw_credit):
        my = lax.axis_index("i")
        cp = cpos_ref[my]
        right = perm_ref[lax.rem(cp + 1, N_DEV)]
        left = perm_ref[lax.rem(cp + N_DEV - 1, N_DEV)]

        barrier = pltpu.get_barrier_semaphore()
        for nbr in (left, right):
            pl.semaphore_signal(
                barrier, inc=1,
                device_id=(nbr,), device_id_type=pl.DeviceIdType.MESH,
            )
        pl.semaphore_wait(barrier, 2)

        w = w_ref[:, :]

        def silu_store(rows, row_start):
            y = jnp.dot(rows, w, preferred_element_type=jnp.float32)
            out_ref[pl.ds(row_start, y.shape[0]), :] = y * jax.nn.sigmoid(y)

        def make_cw(h):
            return pltpu.make_async_remote_copy(
                src_ref=cw_ref.at[h % SLOTS],
                dst_ref=cw_ref.at[(h + 1) % SLOTS],
                send_sem=cw_send.at[h % SLOTS],
                recv_sem=cw_recv.at[(h + 1) % SLOTS],
                device_id=(right,), device_id_type=pl.DeviceIdType.MESH,
            )

        def make_ccw(h):
            return pltpu.make_async_remote_copy(
                src_ref=ccw_ref.at[h % SLOTS],
                dst_ref=ccw_ref.at[(h + 1) % SLOTS],
                send_sem=ccw_send.at[h % SLOTS],
                recv_sem=ccw_recv.at[(h + 1) % SLOTS],
                device_id=(left,), device_id_type=pl.DeviceIdType.MESH,
            )

        cw_ref[0, :, :] = x_ref[:half, :]
        ccw_ref[0, :, :] = x_ref[half:, :]
        cw_desc = [None] * (N_DEV - 1)
        ccw_desc = [None] * (N_DEV - 1)
        cw_desc[0] = make_cw(0)
        ccw_desc[0] = make_ccw(0)
        cw_desc[0].start()
        ccw_desc[0].start()
        silu_store(x_ref[:, :], my * m_per)

        for h in range(N_DEV - 1):
            slot = (h + 1) % SLOTS
            if h >= 1:
                cw_desc[h - 1].wait_send()
                ccw_desc[h - 1].wait_send()
                if h <= N_DEV - 3:
                    pl.semaphore_signal(
                        cw_credit, inc=1,
                        device_id=(left,), device_id_type=pl.DeviceIdType.MESH,
                    )
                    pl.semaphore_signal(
                        ccw_credit, inc=1,
                        device_id=(right,), device_id_type=pl.DeviceIdType.MESH,
                    )
            cw_desc[h].wait_recv()
            ccw_desc[h].wait_recv()
            if h < N_DEV - 2:
                if h + 1 >= 2:
                    pl.semaphore_wait(cw_credit, 1)
                    pl.semaphore_wait(ccw_credit, 1)
                cw_desc[h + 1] = make_cw(h + 1)
                ccw_desc[h + 1] = make_ccw(h + 1)
                cw_desc[h + 1].start()
                ccw_desc[h + 1].start()

            o_cw = perm_ref[lax.rem(cp + N_DEV - 1 - h, N_DEV)]
            o_ccw = perm_ref[lax.rem(cp + 1 + h, N_DEV)]
            silu_store(cw_ref[slot], o_cw * m_per)
            silu_store(ccw_ref[slot], o_ccw * m_per + half)

        cw_desc[N_DEV - 2].wait_send()
        ccw_desc[N_DEV - 2].wait_send()

    out_shape = jax.ShapeDtypeStruct((N_DEV * m_per, n_per), jnp.float32)
    return pl.pallas_call(
        body,
        out_shape=out_shape,
        in_specs=[
            pl.BlockSpec(memory_space=pltpu.SMEM),
            pl.BlockSpec(memory_space=pltpu.SMEM),
            pl.BlockSpec(memory_space=pltpu.VMEM),
            pl.BlockSpec(memory_space=pltpu.VMEM),
        ],
        out_specs=pl.BlockSpec(memory_space=pltpu.VMEM),
        scratch_shapes=[
            pltpu.VMEM((SLOTS, half, k), jnp.float32),
            pltpu.VMEM((SLOTS, half, k), jnp.float32),
            pltpu.SemaphoreType.DMA((SLOTS,)),
            pltpu.SemaphoreType.DMA((SLOTS,)),
            pltpu.SemaphoreType.DMA((SLOTS,)),
            pltpu.SemaphoreType.DMA((SLOTS,)),
            pltpu.SemaphoreType.REGULAR,
            pltpu.SemaphoreType.REGULAR,
        ],
        compiler_params=pltpu.CompilerParams(collective_id=0),
    )(perm_arr, cpos_arr, x, w_mat)


# device time: 1410533 ns/iter; 1.0704x vs baseline; 1.0376x over previous
import jax
import jax.numpy as jnp
from jax import lax
from jax.experimental import pallas as pl
from jax.experimental.pallas import tpu as pltpu

N_DEV = 32
SLOTS = 3
SEGS = 2

PERM = [1, 2, 5, 6, 14, 13, 10, 9, 17, 18, 21, 22, 30, 29, 26, 25,
        24, 27, 28, 31, 23, 20, 19, 16, 8, 11, 12, 15, 7, 4, 3, 0]
CPOS = [0] * N_DEV
for _j, _l in enumerate(PERM):
    CPOS[_l] = _j


def kernel(x, w_mat):
    m_per, k = x.shape
    _, n_per = w_mat.shape
    half = m_per // 2
    seg = half // SEGS

    perm_arr = jnp.asarray(PERM, dtype=jnp.int32)
    cpos_arr = jnp.asarray(CPOS, dtype=jnp.int32)

    def body(perm_ref, cpos_ref, x_ref, w_ref, out_ref, cw_ref, ccw_ref,
             cw_send, cw_recv, ccw_send, ccw_recv, cw_credit, ccw_credit):
        my = lax.axis_index("i")
        cp = cpos_ref[my]
        right = perm_ref[lax.rem(cp + 1, N_DEV)]
        left = perm_ref[lax.rem(cp + N_DEV - 1, N_DEV)]

        barrier = pltpu.get_barrier_semaphore()
        for nbr in (left, right):
            pl.semaphore_signal(
                barrier, inc=1,
                device_id=(nbr,), device_id_type=pl.DeviceIdType.MESH,
            )
        pl.semaphore_wait(barrier, 2)

        w = w_ref[:, :]

        def silu_store(rows, row_start):
            y = jnp.dot(rows, w, preferred_element_type=jnp.float32)
            out_ref[pl.ds(row_start, y.shape[0]), :] = y * jax.nn.sigmoid(y)

        def make(h, g, buf, send_sems, recv_sems, target):
            s, d = h % SLOTS, (h + 1) % SLOTS
            return pltpu.make_async_remote_copy(
                src_ref=buf.at[s, pl.ds(g * seg, seg)],
                dst_ref=buf.at[d, pl.ds(g * seg, seg)],
                send_sem=send_sems.at[s, g],
                recv_sem=recv_sems.at[d, g],
                device_id=(target,), device_id_type=pl.DeviceIdType.MESH,
            )

        def make_cw(h, g):
            return make(h, g, cw_ref, cw_send, cw_recv, right)

        def make_ccw(h, g):
            return make(h, g, ccw_ref, ccw_send, ccw_recv, left)

        cw_ref[0, :, :] = x_ref[:half, :]
        ccw_ref[0, :, :] = x_ref[half:, :]
        cw_desc = [[None, None] for _ in range(N_DEV - 1)]
        ccw_desc = [[None, None] for _ in range(N_DEV - 1)]
        for g in range(SEGS):
            cw_desc[0][g] = make_cw(0, g)
            ccw_desc[0][g] = make_ccw(0, g)
            cw_desc[0][g].start()
            ccw_desc[0][g].start()
        silu_store(x_ref[:, :], my * m_per)

        for h in range(N_DEV - 1):
            slot = (h + 1) % SLOTS
            if h >= 1:
                for g in range(SEGS):
                    cw_desc[h - 1][g].wait_send()
                    ccw_desc[h - 1][g].wait_send()
                if h <= N_DEV - 3:
                    pl.semaphore_signal(
                        cw_credit, inc=1,
                        device_id=(left,), device_id_type=pl.DeviceIdType.MESH,
                    )
                    pl.semaphore_signal(
                        ccw_credit, inc=1,
                        device_id=(right,), device_id_type=pl.DeviceIdType.MESH,
                    )
            fwd = h < N_DEV - 2
            if fwd and h + 1 >= 2:
                pl.semaphore_wait(cw_credit, 1)
                pl.semaphore_wait(ccw_credit, 1)
            for g in range(SEGS):
                cw_desc[h][g].wait_recv()
                if fwd:
                    cw_desc[h + 1][g] = make_cw(h + 1, g)
                    cw_desc[h + 1][g].start()
                ccw_desc[h][g].wait_recv()
                if fwd:
                    ccw_desc[h + 1][g] = make_ccw(h + 1, g)
                    ccw_desc[h + 1][g].start()

            o_cw = perm_ref[lax.rem(cp + N_DEV - 1 - h, N_DEV)]
            o_ccw = perm_ref[lax.rem(cp + 1 + h, N_DEV)]
            silu_store(cw_ref[slot], o_cw * m_per)
            silu_store(ccw_ref[slot], o_ccw * m_per + half)

        for g in range(SEGS):
            cw_desc[N_DEV - 2][g].wait_send()
            ccw_desc[N_DEV - 2][g].wait_send()

    out_shape = jax.ShapeDtypeStruct((N_DEV * m_per, n_per), jnp.float32)
    return pl.pallas_call(
        body,
        out_shape=out_shape,
        in_specs=[
            pl.BlockSpec(memory_space=pltpu.SMEM),
            pl.BlockSpec(memory_space=pltpu.SMEM),
            pl.BlockSpec(memory_space=pltpu.VMEM),
            pl.BlockSpec(memory_space=pltpu.VMEM),
        ],
        out_specs=pl.BlockSpec(memory_space=pltpu.VMEM),
        scratch_shapes=[
            pltpu.VMEM((SLOTS, half, k), jnp.float32),
            pltpu.VMEM((SLOTS, half, k), jnp.float32),
            pltpu.SemaphoreType.DMA((SLOTS, SEGS)),
            pltpu.SemaphoreType.DMA((SLOTS, SEGS)),
            pltpu.SemaphoreType.DMA((SLOTS, SEGS)),
            pltpu.SemaphoreType.DMA((SLOTS, SEGS)),
            pltpu.SemaphoreType.REGULAR,
            pltpu.SemaphoreType.REGULAR,
        ],
        compiler_params=pltpu.CompilerParams(collective_id=0),
    )(perm_arr, cpos_arr, x, w_mat)
